# Initial kernel scaffold; baseline (speedup 1.0000x reference)
import jax
import jax.numpy as jnp
from jax.experimental import pallas as pl


def kernel(h, coord, edge_attr, edge_list, msg_w1, msg_b1, msg_w2, msg_b2, node_w1, node_b1, node_w2, node_b2, edge_w1, edge_b1, edge_w2, edge_b2, rel_w, coord_w1, coord_b1, coord_w2):
    raise NotImplementedError("write your pallas kernel here")


# speedup vs baseline: 31.8473x; 31.8473x over previous
import functools

import numpy as np
import jax
import jax.numpy as jnp
from jax import lax
from jax.experimental import pallas as pl
from jax.experimental.pallas import tpu as pltpu
from jax.experimental.pallas import tpu_sc as plsc

N = 10000
E = 320000
T = 2
D = 128
C = 3
ED = 1

NW = 32
EPT = (T * E) // NW
CH = 80
NCH = EPT // CH
SEPT = E // 16
SNCH = SEPT // CH
NPT = 624


def _silu(x):
    return x * (1.0 / (1.0 + jnp.exp(-x)))


def _make_gather(dims, idx_sel, n_idx):
    n_tab = len(dims)
    mesh = plsc.VectorSubcoreMesh(core_axis_name="c", subcore_axis_name="s")
    out_type = [jax.ShapeDtypeStruct((T * E, d), jnp.float32) for d in dims]
    scratch = (
        [pltpu.VMEM((EPT,), jnp.int32) for _ in range(n_idx)]
        + [pltpu.VMEM((CH, d), jnp.float32) for d in dims]
        + [pltpu.SemaphoreType.DMA, pltpu.SemaphoreType.DMA,
           pltpu.SemaphoreType.DMA]
    )

    @functools.partial(pl.kernel, out_type=out_type, mesh=mesh,
                       scratch_types=scratch)
    def k(*args):
        tabs = args[:n_tab]
        idxs = args[n_tab:n_tab + n_idx]
        outs = args[n_tab + n_idx:n_tab + n_idx + n_tab]
        rest = args[n_tab + n_idx + n_tab:]
        idxv = rest[:n_idx]
        bufs = rest[n_idx:n_idx + n_tab]
        semi, semg, semw = rest[n_idx + n_tab:]

        w = lax.axis_index("s") * 2 + lax.axis_index("c")
        base = w * EPT
        hs = [pltpu.async_copy(idxs[t].at[pl.ds(base, EPT)], idxv[t], semi)
              for t in range(n_idx)]
        for h in hs:
            h.wait()

        def chunk(j, carry):
            off = base + j * CH
            ghs = [
                pltpu.async_copy(
                    tabs[t].at[idxv[idx_sel[t]].at[pl.ds(j * CH, CH)]],
                    bufs[t], semg)
                for t in range(n_tab)
            ]
            for h in ghs:
                h.wait()
            whs = [
                pltpu.async_copy(bufs[t], outs[t].at[pl.ds(off, CH)], semw)
                for t in range(n_tab)
            ]
            for h in whs:
                h.wait()
            return carry

        lax.fori_loop(0, NCH, chunk, 0)

    return k


_gather1 = _make_gather([2 * D, 2 * D], [0, 1], 2)
_gather2 = _make_gather([D, D], [0, 1], 2)


def _make_scatter():
    mesh = plsc.VectorSubcoreMesh(core_axis_name="c", subcore_axis_name="s")
    out_type = jax.ShapeDtypeStruct((T * N, D), jnp.float32)
    scratch = [
        pltpu.VMEM((SNCH, CH), jnp.int32),
        pltpu.VMEM((CH, D), jnp.float32),
        pltpu.VMEM_SHARED((N, D), jnp.float32),
        pltpu.SemaphoreType.DMA,
    ]

    @functools.partial(pl.kernel, out_type=out_type, mesh=mesh,
                       scratch_types=scratch)
    def k(pay, idx1, zh, outh, idxv, bufe, acch, sem):
        c = lax.axis_index("c")
        t = lax.axis_index("s")

        @pl.when(t < 15)
        def _():
            pltpu.async_copy(zh.at[pl.ds(0, NPT)],
                             acch.at[pl.ds(t * NPT, NPT)], sem).wait()

        @pl.when(t == 15)
        def _():
            pltpu.async_copy(zh, acch.at[pl.ds(15 * NPT, 640)], sem).wait()

        plsc.subcore_barrier()

        def chunk(j, carry):
            off = c * E + t * SEPT + j * CH
            a = pltpu.async_copy(pay.at[pl.ds(off, CH)], bufe, sem)
            b = pltpu.async_copy(idx1.at[pl.ds(off, CH)], idxv.at[j], sem)
            a.wait()
            b.wait()
            pltpu.sync_copy(bufe, acch.at[idxv.at[j]], add=True)
            return carry

        lax.fori_loop(0, SNCH, chunk, 0)
        plsc.subcore_barrier()

        @pl.when(t < 15)
        def _():
            pltpu.async_copy(acch.at[pl.ds(t * NPT, NPT)],
                             outh.at[pl.ds(c * N + t * NPT, NPT)], sem).wait()

        @pl.when(t == 15)
        def _():
            pltpu.async_copy(acch.at[pl.ds(15 * NPT, 640)],
                             outh.at[pl.ds(c * N + 15 * NPT, 640)],
                             sem).wait()

    return k


_scatter = _make_scatter()


def _dot(a, b):
    return jnp.dot(a, b, preferred_element_type=jnp.float32)


def _pre_body(h_ref, c16_ref, wr_ref, wc_ref, ta_ref, tb_ref):
    hb = h_ref[...]
    c16 = c16_ref[...]
    z = jnp.zeros((hb.shape[0], 112), jnp.float32)
    ta_ref[...] = jnp.concatenate([_dot(hb, wr_ref[...]), c16, z], axis=1)
    tb_ref[...] = jnp.concatenate([_dot(hb, wc_ref[...]), c16, z], axis=1)


BLKA = 2560
BLK = 1280


def _radial(cd, p1, p2, p3):
    x1 = _dot(cd, p1)
    x2 = _dot(cd, p2)
    return _dot(x1 * x2, p3)


def _k2a_body(c1_ref, c2_ref, p1_ref, p2_ref, p3_ref, cd_ref, ss_ref):
    cd = c1_ref[...][:, :16] - c2_ref[...][:, :16]
    cd_ref[...] = cd
    r = _radial(cd, p1_ref[...], p2_ref[...], p3_ref[...])

    @pl.when(pl.program_id(0) % (E // BLKA) == 0)
    def _():
        ss_ref[...] = jnp.zeros_like(ss_ref)

    ss_ref[...] += jnp.sum(r * r, axis=0).reshape(1, 1, 16)


def _k2b_body(g1_ref, g2_ref, cd_ref, ea_ref, ss_ref, p1_ref, p2_ref,
              p3_ref, wrad_ref, wea_ref, b1_ref, w2_ref, b2_ref, cw1_ref,
              cb1_ref, cw2_ref, b3_ref, ef2_ref, tr_ref):
    cd = cd_ref[...]
    rnorm = jnp.maximum(jnp.sqrt(ss_ref[0]), 1e-12)
    rn = _radial(cd, p1_ref[...], p2_ref[...], p3_ref[...]) / rnorm
    pre = (g1_ref[...] + g2_ref[...] + _dot(rn, wrad_ref[...])
           + ea_ref[...] * wea_ref[...] + b1_ref[...])
    ef = _silu(pre)
    ef2 = _silu(_dot(ef, w2_ref[...]) + b2_ref[...])
    ef2_ref[...] = ef2
    ch = _silu(_dot(ef2, cw1_ref[0]) + cb1_ref[0])
    cm = _dot(ch, cw2_ref[0])
    tcm = _dot(cm, b3_ref[...])
    lane = lax.broadcasted_iota(jnp.int32, tr_ref.shape, 1)
    tr16 = cd * tcm + (lane[:, :16] == 9).astype(jnp.float32)
    tr_ref[...] = jnp.concatenate(
        [tr16, jnp.zeros((tr16.shape[0], 112), jnp.float32)], axis=1)


def _node_body(h_ref, sh0_ref, sh1_ref, st0_ref, st1_ref, c16_ref, rel0_ref,
               rel1_ref, nw1h_ref, nw1a_ref, nb1_ref, nw2_ref, nb2_ref,
               ur_ref, uc_ref, hn_ref, x16_ref, p_ref, q_ref):
    h = h_ref[...]
    agg = _dot(sh0_ref[...], rel0_ref[...]) + _dot(sh1_ref[...], rel1_ref[...])
    pre = _dot(h, nw1h_ref[...]) + _dot(agg, nw1a_ref[...]) + nb1_ref[...]
    hn = h + _dot(_silu(pre), nw2_ref[...]) + nb2_ref[...]
    hn_ref[...] = hn
    ts = st0_ref[...][:, :16] + st1_ref[...][:, :16]
    cnt = ts[:, 9:10]
    x16_ref[...] = c16_ref[...] + ts / jnp.maximum(cnt, 1.0)
    p_ref[...] = _dot(hn, ur_ref[...])
    q_ref[...] = _dot(hn, uc_ref[...])


def _edge_body(e1_ref, e2_ref, ea_ref, uea_ref, eb1_ref, ew2_ref, eb2_ref,
               m_ref):
    eh = _silu(e1_ref[...] + e2_ref[...] + ea_ref[...] * uea_ref[...]
               + eb1_ref[...])
    m_ref[...] = _dot(eh, ew2_ref[...]) + eb2_ref[...]


def _sel_mats():
    p1 = np.zeros((16, 32), np.float32)
    p2 = np.zeros((16, 32), np.float32)
    p3 = np.zeros((32, 16), np.float32)
    b3 = np.zeros((16, 16), np.float32)
    for i in range(3):
        for k in range(3):
            for j in range(3):
                col = 9 * i + 3 * k + j
                p1[3 * i + j, col] = 1.0
                p2[3 * k + j, col] = 1.0
                p3[col, 3 * i + k] = 1.0
        for j in range(3):
            b3[i, 3 * i + j] = 1.0
    return jnp.asarray(p1), jnp.asarray(p2), jnp.asarray(p3), jnp.asarray(b3)


def kernel(h, coord, edge_attr, edge_list, msg_w1, msg_b1, msg_w2, msg_b2,
           node_w1, node_b1, node_w2, node_b2, edge_w1, edge_b1, edge_w2,
           edge_b2, rel_w, coord_w1, coord_b1, coord_w2):
    f32 = jnp.float32
    el = edge_list.astype(jnp.int32)
    rowf = jnp.concatenate([el[0, 0], el[1, 0]])
    colf = jnp.concatenate([el[0, 1], el[1, 1]])
    eaf = edge_attr.reshape(T * E, ED).astype(f32)

    coord16 = jnp.pad(coord.reshape(N, 9), ((0, 0), (0, 7))).astype(f32)
    w_r = msg_w1[:D]
    w_c = msg_w1[D:2 * D]
    w_rad = jnp.pad(msg_w1[2 * D:2 * D + 9], ((0, 7), (0, 0)))
    w_ea = msg_w1[2 * D + 9:2 * D + 10]
    u_r = edge_w1[:D]
    u_ea = edge_w1[D:D + 1]
    u_c = edge_w1[D + 1:]
    p1, p2, p3, b3 = _sel_mats()

    ta, tb = pl.pallas_call(
        _pre_body,
        out_shape=[jax.ShapeDtypeStruct((N, 2 * D), f32)] * 2,
    )(h, coord16, w_r, w_c)

    g1c, g2c = _gather1(ta, tb, rowf, colf)

    nba = (T * E) // BLKA
    cd, ss = pl.pallas_call(
        _k2a_body,
        grid=(nba,),
        in_specs=[
            pl.BlockSpec((BLKA, D), lambda i: (i, 1)),
            pl.BlockSpec((BLKA, D), lambda i: (i, 1)),
            pl.BlockSpec((16, 32), lambda i: (0, 0)),
            pl.BlockSpec((16, 32), lambda i: (0, 0)),
            pl.BlockSpec((32, 16), lambda i: (0, 0)),
        ],
        out_specs=[
            pl.BlockSpec((BLKA, 16), lambda i: (i, 0)),
            pl.BlockSpec((1, 1, 16), lambda i: (i // (E // BLKA), 0, 0)),
        ],
        out_shape=[
            jax.ShapeDtypeStruct((T * E, 16), f32),
            jax.ShapeDtypeStruct((T, 1, 16), f32),
        ],
    )(g1c, g2c, p1, p2, p3)

    nb = E // BLK
    ef2, tr = pl.pallas_call(
        _k2b_body,
        grid=(T * nb,),
        in_specs=[
            pl.BlockSpec((BLK, D), lambda i: (i, 0)),
            pl.BlockSpec((BLK, D), lambda i: (i, 0)),
            pl.BlockSpec((BLK, 16), lambda i: (i, 0)),
            pl.BlockSpec((BLK, ED), lambda i: (i, 0)),
            pl.BlockSpec((1, 1, 16), lambda i: (i // nb, 0, 0)),
            pl.BlockSpec((16, 32), lambda i: (0, 0)),
            pl.BlockSpec((16, 32), lambda i: (0, 0)),
            pl.BlockSpec((32, 16), lambda i: (0, 0)),
            pl.BlockSpec((16, D), lambda i: (0, 0)),
            pl.BlockSpec((1, D), lambda i: (0, 0)),
            pl.BlockSpec((1, D), lambda i: (0, 0)),
            pl.BlockSpec((D, D), lambda i: (0, 0)),
            pl.BlockSpec((1, D), lambda i: (0, 0)),
            pl.BlockSpec((1, D, D), lambda i: (i // nb, 0, 0)),
            pl.BlockSpec((1, 1, D), lambda i: (i // nb, 0, 0)),
            pl.BlockSpec((1, D, 16), lambda i: (i // nb, 0, 0)),
            pl.BlockSpec((16, 16), lambda i: (0, 0)),
        ],
        out_specs=[
            pl.BlockSpec((BLK, D), lambda i: (i, 0)),
            pl.BlockSpec((BLK, D), lambda i: (i, 0)),
        ],
        out_shape=[
            jax.ShapeDtypeStruct((T * E, D), f32),
            jax.ShapeDtypeStruct((T * E, D), f32),
        ],
    )(g1c, g2c, cd, eaf, ss, p1, p2, p3, w_rad, w_ea,
      msg_b1.reshape(1, D), msg_w2, msg_b2.reshape(1, D), coord_w1,
      coord_b1.reshape(T, 1, D),
      jnp.pad(coord_w2, ((0, 0), (0, 0), (0, 13))), b3)

    zeros = jnp.zeros((640, D), f32)
    sh = _scatter(ef2, rowf, zeros)
    st = _scatter(tr, rowf, zeros)

    hn, x16, p, q = pl.pallas_call(
        _node_body,
        out_shape=[
            jax.ShapeDtypeStruct((N, D), f32),
            jax.ShapeDtypeStruct((N, 16), f32),
            jax.ShapeDtypeStruct((N, D), f32),
            jax.ShapeDtypeStruct((N, D), f32),
        ],
    )(h, sh[:N], sh[N:], st[:N], st[N:], coord16, rel_w[0], rel_w[1],
      node_w1[:D], node_w1[D:], node_b1.reshape(1, D), node_w2,
      node_b2.reshape(1, D), u_r, u_c)

    e1, e2 = _gather2(p, q, rowf, colf)

    m = pl.pallas_call(
        _edge_body,
        grid=(T * nb,),
        in_specs=[
            pl.BlockSpec((BLK, D), lambda i: (i, 0)),
            pl.BlockSpec((BLK, D), lambda i: (i, 0)),
            pl.BlockSpec((BLK, ED), lambda i: (i, 0)),
            pl.BlockSpec((1, D), lambda i: (0, 0)),
            pl.BlockSpec((1, D), lambda i: (0, 0)),
            pl.BlockSpec((D, ED), lambda i: (0, 0)),
            pl.BlockSpec((1, ED), lambda i: (0, 0)),
        ],
        out_specs=pl.BlockSpec((BLK, ED), lambda i: (i, 0)),
        out_shape=jax.ShapeDtypeStruct((T * E, ED), f32),
    )(e1, e2, eaf, u_ea, edge_b1.reshape(1, D), edge_w2,
      edge_b2.reshape(1, ED))

    x_new = x16[:, :9].reshape(N, C, 3)
    return hn, x_new, m.reshape(T, E, ED)
